# baseline (speedup 1.0000x reference)
import functools

import jax
import jax.numpy as jnp
from jax import lax
from jax.experimental import pallas as pl
from jax.experimental.pallas import tpu as pltpu
from jax.experimental.pallas import tpu_sc as plsc

_M = 7
_TAU = 3


def kernel(x):
    B, S, C = x.shape
    nwin = _M + 1
    valid_t = S - _M * _TAU
    half = (valid_t + 1) // 2

    mesh = plsc.VectorSubcoreMesh(core_axis_name="c", subcore_axis_name="s")

    @functools.partial(
        pl.kernel,
        mesh=mesh,
        out_type=jax.ShapeDtypeStruct((B, valid_t, nwin, C), jnp.float32),
        compiler_params=pltpu.CompilerParams(use_tc_tiling_on_sc=False),
    )
    def run(x_hbm, out_hbm):
        c = lax.axis_index("c")
        s = lax.axis_index("s")
        wid = s * 2 + c
        b = wid // (2 * nwin)
        i = (wid // 2) % nwin
        h = wid % 2
        t0 = jnp.where(h == 0, 0, valid_t - half)
        pltpu.sync_copy(
            x_hbm.at[b, pl.ds(t0 + i * _TAU, half), :],
            out_hbm.at[b, pl.ds(t0, half), i, :],
        )

    out = run(x)
    return out.reshape(B, valid_t, nwin * C)


# speedup vs baseline: 9.2625x; 9.2625x over previous
import functools

import jax
import jax.numpy as jnp
from jax import lax
from jax.experimental import pallas as pl
from jax.experimental.pallas import tpu as pltpu
from jax.experimental.pallas import tpu_sc as plsc

_M = 7
_TAU = 3


def kernel(x):
    B, S, C = x.shape
    nwin = _M + 1
    valid_t = S - _M * _TAU
    half = (valid_t + 1) // 2
    nch = 13
    tck = half // nch
    assert tck * nch == half

    mesh = plsc.VectorSubcoreMesh(core_axis_name="c", subcore_axis_name="s")

    @functools.partial(
        pl.kernel,
        mesh=mesh,
        out_type=jax.ShapeDtypeStruct((B, valid_t, nwin, C), jnp.float32),
        scratch_types=[
            pltpu.VMEM((2, tck, C), jnp.float32),
            pltpu.SemaphoreType.DMA,
            pltpu.SemaphoreType.DMA,
        ],
        compiler_params=pltpu.CompilerParams(use_tc_tiling_on_sc=False),
    )
    def run(x_hbm, out_hbm, buf, gsem, ssem):
        c = lax.axis_index("c")
        s = lax.axis_index("s")
        wid = s * 2 + c
        b = wid // (2 * nwin)
        i = (wid // 2) % nwin
        h = wid % 2
        t0 = jnp.where(h == 0, 0, valid_t - half)

        def gather_start(k):
            return pltpu.async_copy(
                x_hbm.at[b, pl.ds(t0 + i * _TAU + k * tck, tck), :],
                buf.at[k % 2],
                gsem,
            )

        def scatter_start(k):
            return pltpu.async_copy(
                buf.at[k % 2],
                out_hbm.at[b, pl.ds(t0 + k * tck, tck), i, :],
                ssem,
            )

        gops = {}
        sops = {}
        gops[0] = gather_start(0)
        for k in range(nch):
            gops[k].wait()
            sops[k] = scatter_start(k)
            if k + 1 < nch:
                if k >= 1:
                    sops[k - 1].wait()
                gops[k + 1] = gather_start(k + 1)
        sops[nch - 2].wait()
        sops[nch - 1].wait()

    out = run(x)
    return out.reshape(B, valid_t, nwin * C)


# speedup vs baseline: 10.4858x; 1.1321x over previous
import functools

import jax
import jax.numpy as jnp
from jax import lax
from jax.experimental import pallas as pl
from jax.experimental.pallas import tpu as pltpu
from jax.experimental.pallas import tpu_sc as plsc

_M = 7
_TAU = 3


def kernel(x):
    B, S, C = x.shape
    nwin = _M + 1
    halo = _M * _TAU
    valid_t = S - halo
    tsub = 64
    nsub = 2
    ttile = tsub * nsub
    nblk = 16
    ext = tsub + halo
    mesh = plsc.VectorSubcoreMesh(core_axis_name="c", subcore_axis_name="s")

    @functools.partial(
        pl.kernel,
        mesh=mesh,
        out_type=jax.ShapeDtypeStruct((B, valid_t, nwin, C), jnp.float32),
        scratch_types=[
            pltpu.VMEM((nsub, ext, C), jnp.float32),
            pltpu.SemaphoreType.DMA,
            pltpu.SemaphoreType.DMA,
        ],
        compiler_params=pltpu.CompilerParams(use_tc_tiling_on_sc=False),
    )
    def run(x_hbm, out_hbm, buf, gsem, ssem):
        c = lax.axis_index("c")
        s = lax.axis_index("s")
        wid = s * 2 + c
        b = wid // nblk
        j = wid % nblk
        tbase = jnp.minimum(j * (ttile - 1), valid_t - ttile)

        def gather_start(k):
            return pltpu.async_copy(
                x_hbm.at[b, pl.ds(tbase + k * tsub, ext), :],
                buf.at[k],
                gsem,
            )

        gops = {}
        sops = []
        gops[0] = gather_start(0)
        for k in range(nsub):
            if k + 1 < nsub:
                gops[k + 1] = gather_start(k + 1)
            gops[k].wait()
            for i in range(nwin):
                sops.append(
                    pltpu.async_copy(
                        buf.at[k, pl.ds(i * _TAU, tsub), :],
                        out_hbm.at[b, pl.ds(tbase + k * tsub, tsub), i, :],
                        ssem,
                    )
                )
        for op in sops:
            op.wait()

    out = run(x)
    return out.reshape(B, valid_t, nwin * C)
